# baseline (speedup 1.0000x reference)
import functools

import jax
import jax.numpy as jnp
from jax import lax
from jax.experimental import pallas as pl
from jax.experimental.pallas import tpu as pltpu
from jax.experimental.pallas import tpu_sc as plsc

NC = 2
NS = 16
NW = NC * NS
LANES = 16


CH = 4608
CHH = CH // 2
VMAIN = 999936
VTAIL = 64


def _project_table(table, W):
    V, E = table.shape
    OUT = W.shape[0]
    nmain = VMAIN // CH
    ng = nmain + 1
    tail = jnp.reshape(
        lax.slice(table, (VMAIN, 0), (V, 0 + E)) @ W.T, (VTAIL // 2, 2 * OUT)
    )

    def body(tt_ref, wt_ref, tail_ref, o_ref, lo0, lo1, sem0, sem1):
        i = pl.program_id(0)
        los = [lo0, lo1]
        sems = [sem0, sem1]

        def cp(j, b):
            return pltpu.make_async_copy(
                tt_ref.at[:, pl.ds(CH * j, CH)], los[b], sems[b]
            )

        @pl.when(i == 0)
        def _():
            cp(0, 0).start()

        def work(b):
            @pl.when(i + 1 < nmain)
            def _():
                cp(i + 1, 1 - b).start()
            cp(i, b).wait()
            res = lax.dot_general(
                los[b][...], wt_ref[...],
                (((0,), (0,)), ((), ())),
                preferred_element_type=jnp.float32,
            )
            o_ref[...] = jnp.concatenate(
                [res[:CHH], res[CHH:]], axis=1
            )

        @pl.when(jnp.logical_and(i % 2 == 0, i < nmain))
        def _():
            work(0)

        @pl.when(jnp.logical_and(i % 2 == 1, i < nmain))
        def _():
            work(1)

        @pl.when(i == nmain)
        def _():
            o_ref[pl.ds(0, VTAIL // 2), :] = tail_ref[...]
            o_ref[pl.ds(VTAIL // 2, CHH - VTAIL // 2), :] = jnp.zeros(
                (CHH - VTAIL // 2, 2 * OUT), jnp.float32
            )

    y = pl.pallas_call(
        body,
        grid=(ng,),
        in_specs=[
            pl.BlockSpec(memory_space=pl.ANY),
            pl.BlockSpec((E, OUT), lambda i: (0, 0)),
            pl.BlockSpec((VTAIL // 2, 2 * OUT), lambda i: (0, 0)),
        ],
        out_specs=pl.BlockSpec((CHH, 2 * OUT), lambda i: (i, 0)),
        out_shape=jax.ShapeDtypeStruct((ng * CHH, 2 * OUT), jnp.float32),
        scratch_shapes=[
            pltpu.VMEM((E, CH), jnp.float32),
            pltpu.VMEM((E, CH), jnp.float32),
            pltpu.SemaphoreType.DMA,
            pltpu.SemaphoreType.DMA,
        ],
    )(table.T, W.T, tail)
    return jnp.reshape(y, (ng * CH, OUT))


def _make_bag_kernel(B, L, E, V, interpret=False):
    assert L % 2 == 0
    Lh = L // 2
    bpw = B // NW
    CB = 8
    nchunks = bpw // CB
    assert bpw % CB == 0
    nseg = 2 * CB

    mesh = plsc.VectorSubcoreMesh(
        core_axis_name="c", subcore_axis_name="s", num_cores=NC, num_subcores=NS
    )

    @functools.partial(
        pl.kernel,
        out_type=jax.ShapeDtypeStruct((B, E), jnp.float32),
        mesh=mesh,
        scratch_types=[
            pltpu.VMEM((nseg, Lh), jnp.int32),
            pltpu.VMEM((nseg, Lh, E), jnp.float32),
            pltpu.VMEM((CB, E), jnp.float32),
            pltpu.SemaphoreType.DMA,
        ],
        compiler_params=pltpu.CompilerParams(use_tc_tiling_on_sc=False),
        interpret=interpret,
    )
    def bag_kernel(x_hbm, table_hbm, bag_hbm, idx_v, rows_v, bag_v, sem):
        wid = lax.axis_index("s") * NC + lax.axis_index("c")
        base = wid * bpw
        scale = jnp.float32(1.0 / L)
        chh = jnp.full((LANES,), CHH, jnp.int32)
        vmain = jnp.full((LANES,), VMAIN, jnp.int32)
        lane = lax.iota(jnp.int32, LANES)
        nfull = Lh // LANES
        tail0 = Lh - LANES
        ntrans = nfull * LANES - tail0

        def permute_idx():
            def perm(t):
                n = jnp.right_shift(t, 9)
                j = jnp.right_shift(n * 29128, 18)
                base = j * (-CH) + t + t
                c = t - j * CH
                v = jnp.where(c < chh, base, base - (CH - 1))
                return jnp.where(t < vmain, v, t)

            def row(s, carry):
                def one(i, c2):
                    t = idx_v[s, pl.ds(LANES * i, LANES)]
                    idx_v[s, pl.ds(LANES * i, LANES)] = perm(t)
                    return c2
                lax.fori_loop(0, nfull, one, 0, unroll=2)
                if tail0 % LANES:
                    t = idx_v[s, pl.ds(tail0, LANES)]
                    idx_v[s, pl.ds(tail0, LANES)] = jnp.where(
                        lane < ntrans, t, perm(t)
                    )
                return carry

            lax.fori_loop(0, nseg, row, 0)

        def chunk(ci, carry):
            off = base + ci * CB
            pltpu.sync_copy(x_hbm.at[pl.ds(2 * off, nseg)], idx_v)
            permute_idx()
            cps = [
                pltpu.async_copy(table_hbm.at[idx_v.at[s]], rows_v.at[s], sem)
                for s in range(nseg)
            ]
            for cp in cps:
                cp.wait()
            for r in range(CB):
                def red(j, acc):
                    return tuple(
                        acc[c]
                        + rows_v[2 * r, j, pl.ds(LANES * c, LANES)]
                        + rows_v[2 * r + 1, j, pl.ds(LANES * c, LANES)]
                        for c in range(E // LANES)
                    )
                acc0 = tuple(
                    jnp.zeros((LANES,), jnp.float32) for _ in range(E // LANES)
                )
                acc = lax.fori_loop(0, Lh, red, acc0, unroll=2)
                for c in range(E // LANES):
                    bag_v[r, pl.ds(LANES * c, LANES)] = acc[c] * scale
            pltpu.sync_copy(bag_v, bag_hbm.at[pl.ds(off, CB)])
            return carry

        lax.fori_loop(0, nchunks, chunk, 0)

    return bag_kernel


def _bias_bcast(bag, b2, YLEN):
    B, OUT = bag.shape
    BT = 512

    def body(bag_ref, b_ref, out_ref):
        pot = bag_ref[...].T + b_ref[...]
        out_ref[...] = jnp.broadcast_to(pot[None, :, :], (YLEN, OUT, BT))

    return pl.pallas_call(
        body,
        grid=(B // BT,),
        in_specs=[
            pl.BlockSpec((BT, OUT), lambda i: (i, 0)),
            pl.BlockSpec((OUT, 1), lambda i: (0, 0)),
        ],
        out_specs=pl.BlockSpec((YLEN, OUT, BT), lambda i: (0, 0, i)),
        out_shape=jax.ShapeDtypeStruct((YLEN, OUT, B), jnp.float32),
    )(bag, b2)


def kernel(x, y_c, table, W, b):
    B, L = x.shape
    YLEN = y_c.shape[1]
    V, E = table.shape
    OUT = W.shape[0]
    ptable = _project_table(table, W)
    x_r = x.astype(jnp.int32).reshape(2 * B, L // 2)
    bag = _make_bag_kernel(B, L, OUT, V)(x_r, ptable)
    out = _bias_bcast(bag, b.reshape(OUT, 1), YLEN)
    return jnp.transpose(out, (2, 0, 1))


# speedup vs baseline: 1.2078x; 1.2078x over previous
import functools

import jax
import jax.numpy as jnp
from jax import lax
from jax.experimental import pallas as pl
from jax.experimental.pallas import tpu as pltpu
from jax.experimental.pallas import tpu_sc as plsc

NC = 2
NS = 16
NW = NC * NS
LANES = 16


CH = 16128
CHH = CH // 2
VMAIN = 999936
VTAIL = 64


def _project_table(table, W):
    V, E = table.shape
    OUT = W.shape[0]
    nmain = VMAIN // CH
    ng = nmain + 1
    tail = jnp.reshape(
        lax.slice(table, (VMAIN, 0), (V, 0 + E)) @ W.T, (VTAIL // 2, 2 * OUT)
    )

    def body(tt_ref, wt_ref, tail_ref, o_ref, lo0, lo1, sem0, sem1):
        i = pl.program_id(0)
        los = [lo0, lo1]
        sems = [sem0, sem1]

        def cp(j, b):
            return pltpu.make_async_copy(
                tt_ref.at[:, pl.ds(CH * j, CH)], los[b], sems[b]
            )

        @pl.when(i == 0)
        def _():
            cp(0, 0).start()

        def work(b):
            @pl.when(i + 1 < nmain)
            def _():
                cp(i + 1, 1 - b).start()
            cp(i, b).wait()
            res = lax.dot_general(
                los[b][...], wt_ref[...],
                (((0,), (0,)), ((), ())),
                preferred_element_type=jnp.float32,
            )
            o_ref[...] = jnp.concatenate(
                [res[:CHH], res[CHH:]], axis=1
            )

        @pl.when(jnp.logical_and(i % 2 == 0, i < nmain))
        def _():
            work(0)

        @pl.when(jnp.logical_and(i % 2 == 1, i < nmain))
        def _():
            work(1)

        @pl.when(i == nmain)
        def _():
            o_ref[pl.ds(0, VTAIL // 2), :] = tail_ref[...]
            o_ref[pl.ds(VTAIL // 2, CHH - VTAIL // 2), :] = jnp.zeros(
                (CHH - VTAIL // 2, 2 * OUT), jnp.float32
            )

    y = pl.pallas_call(
        body,
        grid=(ng,),
        in_specs=[
            pl.BlockSpec(memory_space=pl.ANY),
            pl.BlockSpec((E, OUT), lambda i: (0, 0)),
            pl.BlockSpec((VTAIL // 2, 2 * OUT), lambda i: (0, 0)),
        ],
        out_specs=pl.BlockSpec((CHH, 2 * OUT), lambda i: (i, 0)),
        out_shape=jax.ShapeDtypeStruct((ng * CHH, 2 * OUT), jnp.float32),
        scratch_shapes=[
            pltpu.VMEM((E, CH), jnp.float32),
            pltpu.VMEM((E, CH), jnp.float32),
            pltpu.SemaphoreType.DMA,
            pltpu.SemaphoreType.DMA,
        ],
    )(table.T, W.T, tail)
    return jnp.reshape(y, (ng * CH, OUT))


def _make_bag_kernel(B, L, E, V, interpret=False):
    assert L % 2 == 0
    Lh = L // 2
    bpw = B // NW
    CB = 8
    nchunks = bpw // CB
    assert bpw % CB == 0
    nseg = 2 * CB

    mesh = plsc.VectorSubcoreMesh(
        core_axis_name="c", subcore_axis_name="s", num_cores=NC, num_subcores=NS
    )

    @functools.partial(
        pl.kernel,
        out_type=jax.ShapeDtypeStruct((B, E), jnp.float32),
        mesh=mesh,
        scratch_types=[
            pltpu.VMEM((nseg, Lh), jnp.int32),
            pltpu.VMEM((nseg, Lh, E), jnp.float32),
            pltpu.VMEM((CB, E), jnp.float32),
            pltpu.SemaphoreType.DMA,
        ],
        compiler_params=pltpu.CompilerParams(use_tc_tiling_on_sc=False),
        interpret=interpret,
    )
    def bag_kernel(x_hbm, table_hbm, bag_hbm, idx_v, rows_v, bag_v, sem):
        wid = lax.axis_index("s") * NC + lax.axis_index("c")
        base = wid * bpw
        scale = jnp.float32(1.0 / L)
        chh = jnp.full((LANES,), CHH, jnp.int32)
        vmain = jnp.full((LANES,), VMAIN, jnp.int32)
        lane = lax.iota(jnp.int32, LANES)
        nfull = Lh // LANES
        tail0 = Lh - LANES
        ntrans = nfull * LANES - tail0

        def permute_idx():
            def perm(t):
                n = jnp.right_shift(t, 7)
                j = jnp.right_shift(n * 33289, 22)
                base = j * (-CH) + t + t
                c = t - j * CH
                v = jnp.where(c < chh, base, base - (CH - 1))
                return jnp.where(t < vmain, v, t)

            def row(s, carry):
                def one(i, c2):
                    t = idx_v[s, pl.ds(LANES * i, LANES)]
                    idx_v[s, pl.ds(LANES * i, LANES)] = perm(t)
                    return c2
                lax.fori_loop(0, nfull, one, 0, unroll=2)
                if tail0 % LANES:
                    t = idx_v[s, pl.ds(tail0, LANES)]
                    idx_v[s, pl.ds(tail0, LANES)] = jnp.where(
                        lane < ntrans, t, perm(t)
                    )
                return carry

            lax.fori_loop(0, nseg, row, 0)

        def chunk(ci, carry):
            off = base + ci * CB
            pltpu.sync_copy(x_hbm.at[pl.ds(2 * off, nseg)], idx_v)
            permute_idx()
            cps = [
                pltpu.async_copy(table_hbm.at[idx_v.at[s]], rows_v.at[s], sem)
                for s in range(nseg)
            ]
            for cp in cps:
                cp.wait()
            for r in range(CB):
                def red(j, acc):
                    return tuple(
                        acc[c]
                        + rows_v[2 * r, j, pl.ds(LANES * c, LANES)]
                        + rows_v[2 * r + 1, j, pl.ds(LANES * c, LANES)]
                        for c in range(E // LANES)
                    )
                acc0 = tuple(
                    jnp.zeros((LANES,), jnp.float32) for _ in range(E // LANES)
                )
                acc = lax.fori_loop(0, Lh, red, acc0, unroll=2)
                for c in range(E // LANES):
                    bag_v[r, pl.ds(LANES * c, LANES)] = acc[c] * scale
            pltpu.sync_copy(bag_v, bag_hbm.at[pl.ds(off, CB)])
            return carry

        lax.fori_loop(0, nchunks, chunk, 0)

    return bag_kernel


def _bias_bcast(bag, b2, YLEN):
    B, OUT = bag.shape
    BT = 512

    def body(bag_ref, b_ref, out_ref):
        pot = bag_ref[...].T + b_ref[...]
        out_ref[...] = jnp.broadcast_to(pot[None, :, :], (YLEN, OUT, BT))

    return pl.pallas_call(
        body,
        grid=(B // BT,),
        in_specs=[
            pl.BlockSpec((BT, OUT), lambda i: (i, 0)),
            pl.BlockSpec((OUT, 1), lambda i: (0, 0)),
        ],
        out_specs=pl.BlockSpec((YLEN, OUT, BT), lambda i: (0, 0, i)),
        out_shape=jax.ShapeDtypeStruct((YLEN, OUT, B), jnp.float32),
    )(bag, b2)


def kernel(x, y_c, table, W, b):
    B, L = x.shape
    YLEN = y_c.shape[1]
    V, E = table.shape
    OUT = W.shape[0]
    ptable = _project_table(table, W)
    x_r = x.astype(jnp.int32).reshape(2 * B, L // 2)
    bag = _make_bag_kernel(B, L, OUT, V)(x_r, ptable)
    out = _bias_bcast(bag, b.reshape(OUT, 1), YLEN)
    return jnp.transpose(out, (2, 0, 1))
